# baseline (speedup 1.0000x reference)
import functools

import jax
import jax.numpy as jnp
from jax import lax
from jax.experimental import pallas as pl
from jax.experimental.pallas import tpu as pltpu
from jax.experimental.pallas import tpu_sc as plsc

N_TC = 2
N_TILE = 16
NW = N_TC * N_TILE
NSEG = 1000
TRASH = 1000
ACC_ROWS = 1024
RPT = ACC_ROWS // N_TILE
D = 128
G = 512
GB = G // 128


def _make_sc_segsum(chunk, compact):
    nv = -(-chunk // 16)
    stage_cap = nv * 16 + 16
    if compact:
        cap = (-(-chunk // G) + 1) * G
    else:
        cap = -(-chunk // G) * G
        assert cap <= stage_cap
    mesh = plsc.VectorSubcoreMesh(core_axis_name="c", subcore_axis_name="s")

    @functools.partial(
        pl.kernel,
        mesh=mesh,
        compiler_params=pltpu.CompilerParams(needs_layout_passes=False),
        out_type=(
            jax.ShapeDtypeStruct((N_TC, ACC_ROWS, D), jnp.float32),
            jax.ShapeDtypeStruct((N_TC, ACC_ROWS, D), jnp.float32),
        ),
        scratch_types=[
            pltpu.VMEM((stage_cap,), jnp.int32),
            pltpu.VMEM((stage_cap,), jnp.int32),
            pltpu.VMEM((cap,), jnp.int32),
            pltpu.VMEM((cap,), jnp.int32),
            pltpu.VMEM((GB, 128), jnp.int32),
            pltpu.VMEM((G, D), jnp.float32),
            pltpu.VMEM((128, D), jnp.float32),
            pltpu.VMEM_SHARED((ACC_ROWS, D), jnp.float32),
            pltpu.VMEM_SHARED((ACC_ROWS, D), jnp.float32),
            pltpu.SemaphoreType.DMA,
            pltpu.SemaphoreType.DMA,
            pltpu.SemaphoreType.DMA,
        ],
    )
    def seg_kernel(edges, table, zrow, ones_h,
                   pout, cout,
                   src_st, dst_st, csrc_c, cdst_c, idx4, rows_g, ones_v,
                   acc, cacc, sem0, sem1, sem2):
        cid = lax.axis_index("c")
        sid = lax.axis_index("s")
        w = sid * N_TC + cid

        pltpu.sync_copy(zrow, acc.at[pl.ds(sid * RPT, RPT)])
        pltpu.sync_copy(zrow, cacc.at[pl.ds(sid * RPT, RPT)])
        pltpu.sync_copy(ones_h, ones_v)
        pltpu.sync_copy(edges.at[pl.ds(w * chunk, chunk)],
                        src_st.at[pl.ds(0, chunk)])
        pltpu.sync_copy(edges.at[pl.ds(NW * chunk + w * chunk, chunk)],
                        dst_st.at[pl.ds(0, chunk)])
        plsc.subcore_barrier()

        pad_s = jnp.zeros((16,), jnp.int32)
        pad_d = jnp.full((16,), TRASH, jnp.int32)

        if compact:
            csrc, cdst = csrc_c, cdst_c
            src_st[pl.ds(chunk, 16)] = pad_s
            dst_st[pl.ds(chunk, 16)] = pad_d
            src_st[pl.ds(chunk + 8, 16)] = pad_s
            dst_st[pl.ds(chunk + 8, 16)] = pad_d

            lane = lax.iota(jnp.int32, 16)

            def cstep(i, offv):
                b0 = i * 32
                sv0 = src_st[pl.ds(b0, 16)]
                dv0 = dst_st[pl.ds(b0, 16)]
                sv1 = src_st[pl.ds(b0 + 16, 16)]
                dv1 = dst_st[pl.ds(b0 + 16, 16)]
                m0 = dv0 < NSEG
                m1 = dv1 < NSEG
                k0, s0, _ = plsc.sort_key_val(dv0, sv0, mask=m0)
                k1, s1, _ = plsc.sort_key_val(dv1, sv1, mask=m1)
                n0 = plsc.all_reduce_population_count(m0)
                n1 = plsc.all_reduce_population_count(m1)
                p0 = offv + lane
                plsc.store_scatter(cdst, [p0], k0)
                plsc.store_scatter(csrc, [p0], s0)
                p1 = p0 + n0
                plsc.store_scatter(cdst, [p1], k1)
                plsc.store_scatter(csrc, [p1], s1)
                return offv + (n0 + n1)

            offv = lax.fori_loop(0, (nv + 1) // 2, cstep,
                                 jnp.zeros((16,), jnp.int32))
            off = offv[0]

            def pstep(k, _):
                csrc[pl.ds(off + k * 16, 16)] = pad_s
                cdst[pl.ds(off + k * 16, 16)] = pad_d
                return 0

            lax.fori_loop(0, G // 16, pstep, 0)
            ng = (off + (G - 1)) // G
        else:
            csrc, cdst = src_st, dst_st
            for p in sorted({min(q, cap - 16) for q in range(chunk, cap, 16)}):
                csrc[pl.ds(p, 16)] = pad_s
                cdst[pl.ds(p, 16)] = pad_d
            ng = cap // G

        def group(g, _):
            for b in range(GB):
                def icp(k, _):
                    idx4[b, pl.ds(k * 16, 16)] = \
                        cdst[pl.ds(g * G + b * 128 + k * 16, 16)]
                    return 0

                lax.fori_loop(0, 8, icp, 0)
            pltpu.async_copy(table.at[csrc.at[pl.ds(g * G, G)]],
                             rows_g, sem0).wait()
            for b in range(GB):
                pltpu.async_copy(rows_g.at[pl.ds(b * 128, 128)],
                                 acc.at[idx4.at[b]], sem1, add=True)
                pltpu.async_copy(ones_v, cacc.at[idx4.at[b]], sem2, add=True)
            for b in range(GB):
                pltpu.make_async_copy(rows_g.at[pl.ds(b * 128, 128)],
                                      acc.at[idx4.at[b]], sem1).wait()
                pltpu.make_async_copy(ones_v, cacc.at[idx4.at[b]], sem2).wait()
            return 0

        lax.fori_loop(0, ng, group, 0)
        plsc.subcore_barrier()

        r0 = sid * RPT
        pltpu.sync_copy(acc.at[pl.ds(r0, RPT)], pout.at[cid, pl.ds(r0, RPT)])
        pltpu.sync_copy(cacc.at[pl.ds(r0, RPT)], cout.at[cid, pl.ds(r0, RPT)])

    return seg_kernel


_sc_seg0 = _make_sc_segsum(160000 // NW, compact=True)
_sc_seg1 = _make_sc_segsum(32000 // NW, compact=False)


def _tc_body(pref, cref, xref, wlref, wrref, blref, brref, oref, *, act):
    agg = pref[0, :NSEG, :] + pref[1, :NSEG, :]
    cnt = cref[0, :NSEG, 0:1] + cref[1, :NSEG, 0:1]
    mean = agg / jnp.maximum(cnt, 1.0)
    z = (jnp.dot(mean, wlref[...], preferred_element_type=jnp.float32)
         + jnp.dot(xref[...], wrref[...], preferred_element_type=jnp.float32)
         + blref[...] + brref[...])
    if act == "relu":
        oref[...] = jnp.maximum(z, 0.0)
    else:
        m = jnp.max(z, axis=-1, keepdims=True)
        e = jnp.exp(z - m)
        oref[...] = z - m - jnp.log(jnp.sum(e, axis=-1, keepdims=True))


def _tc_layer(p, c, xt, wl, wr, bl, br, act):
    return pl.pallas_call(
        functools.partial(_tc_body, act=act),
        out_shape=jax.ShapeDtypeStruct((NSEG, D), jnp.float32),
    )(p, c, xt, wl, wr, bl.reshape(1, D), br.reshape(1, D))


def kernel(x, edge_index0, edge_index1, W_l0, b_l0, W_r0, b_r0,
           W_l1, b_l1, W_r1, b_r1):
    e0 = edge_index0.astype(jnp.int32).reshape(-1)
    e1 = edge_index1.astype(jnp.int32).reshape(-1)
    zrow = jnp.zeros((RPT, D), jnp.float32)
    ones_h = jnp.ones((128, D), jnp.float32)

    p0, c0 = _sc_seg0(e0, x, zrow, ones_h)
    h = _tc_layer(p0, c0, x[:NSEG], W_l0, W_r0, b_l0, b_r0, "relu")
    p1, c1 = _sc_seg1(e1, h, zrow, ones_h)
    return _tc_layer(p1, c1, h, W_l1, W_r1, b_l1, b_r1, "logsoftmax")


# speedup vs baseline: 1.6124x; 1.6124x over previous
import functools

import jax
import jax.numpy as jnp
from jax import lax
from jax.experimental import pallas as pl
from jax.experimental.pallas import tpu as pltpu
from jax.experimental.pallas import tpu_sc as plsc

N_TC = 2
N_TILE = 16
NW = N_TC * N_TILE
NSEG = 1000
TRASH = 1000
ACC_ROWS = 1024
RPT = ACC_ROWS // N_TILE
D = 128


def _make_sc_segsum(chunk, compact):
    nv = -(-chunk // 16)
    stage_cap = nv * 16 + 16
    if compact:
        cap = (-(-chunk // 128) + 1) * 128
    else:
        cap = -(-chunk // 128) * 128
        assert cap <= stage_cap
    mesh = plsc.VectorSubcoreMesh(core_axis_name="c", subcore_axis_name="s")

    @functools.partial(
        pl.kernel,
        mesh=mesh,
        compiler_params=pltpu.CompilerParams(needs_layout_passes=False),
        out_type=(
            jax.ShapeDtypeStruct((N_TC, ACC_ROWS, D), jnp.float32),
            jax.ShapeDtypeStruct((N_TC, ACC_ROWS, D), jnp.float32),
        ),
        scratch_types=[
            pltpu.VMEM((stage_cap,), jnp.int32),
            pltpu.VMEM((stage_cap,), jnp.int32),
            pltpu.VMEM((cap,), jnp.int32),
            pltpu.VMEM((cap,), jnp.int32),
            pltpu.VMEM((2, 128), jnp.int32),
            pltpu.VMEM((2, 128, D), jnp.float32),
            pltpu.VMEM((128, D), jnp.float32),
            pltpu.VMEM_SHARED((ACC_ROWS, D), jnp.float32),
            pltpu.VMEM_SHARED((ACC_ROWS, D), jnp.float32),
            pltpu.SemaphoreType.DMA,
            pltpu.SemaphoreType.DMA,
        ],
    )
    def seg_kernel(edges, table, zrow, ones_h,
                   pout, cout,
                   src_st, dst_st, csrc_c, cdst_c, idx_blk, rows_v, ones_v,
                   acc, cacc, sem_g0, sem_g1):
        cid = lax.axis_index("c")
        sid = lax.axis_index("s")
        w = sid * N_TC + cid

        pltpu.sync_copy(zrow, acc.at[pl.ds(sid * RPT, RPT)])
        pltpu.sync_copy(zrow, cacc.at[pl.ds(sid * RPT, RPT)])
        pltpu.sync_copy(ones_h, ones_v)
        pltpu.sync_copy(edges.at[pl.ds(w * chunk, chunk)],
                        src_st.at[pl.ds(0, chunk)])
        pltpu.sync_copy(edges.at[pl.ds(NW * chunk + w * chunk, chunk)],
                        dst_st.at[pl.ds(0, chunk)])
        plsc.subcore_barrier()

        pad_s = jnp.zeros((16,), jnp.int32)
        pad_d = jnp.full((16,), TRASH, jnp.int32)

        if compact:
            csrc, cdst = csrc_c, cdst_c
            src_st[pl.ds(chunk, 16)] = pad_s
            dst_st[pl.ds(chunk, 16)] = pad_d
            src_st[pl.ds(chunk + 8, 16)] = pad_s
            dst_st[pl.ds(chunk + 8, 16)] = pad_d

            lane = lax.iota(jnp.int32, 16)

            def cstep(i, offv):
                b0 = i * 32
                sv0 = src_st[pl.ds(b0, 16)]
                dv0 = dst_st[pl.ds(b0, 16)]
                sv1 = src_st[pl.ds(b0 + 16, 16)]
                dv1 = dst_st[pl.ds(b0 + 16, 16)]
                m0 = dv0 < NSEG
                m1 = dv1 < NSEG
                k0, s0, _ = plsc.sort_key_val(dv0, sv0, mask=m0)
                k1, s1, _ = plsc.sort_key_val(dv1, sv1, mask=m1)
                n0 = plsc.all_reduce_population_count(m0)
                n1 = plsc.all_reduce_population_count(m1)
                p0 = offv + lane
                plsc.store_scatter(cdst, [p0], k0)
                plsc.store_scatter(csrc, [p0], s0)
                p1 = p0 + n0
                plsc.store_scatter(cdst, [p1], k1)
                plsc.store_scatter(csrc, [p1], s1)
                return offv + (n0 + n1)

            offv = lax.fori_loop(0, (nv + 1) // 2, cstep,
                                 jnp.zeros((16,), jnp.int32))
            off = offv[0]

            def pstep(k, _):
                csrc[pl.ds(off + k * 16, 16)] = pad_s
                cdst[pl.ds(off + k * 16, 16)] = pad_d
                return 0

            lax.fori_loop(0, 8, pstep, 0)
            nb = (off + 127) // 128
        else:
            csrc, cdst = src_st, dst_st
            for p in sorted({min(q, cap - 16) for q in range(chunk, cap, 16)}):
                csrc[pl.ds(p, 16)] = pad_s
                cdst[pl.ds(p, 16)] = pad_d
            nb = cap // 128

        def stage_block(j, b, sem):
            def icp(k, _):
                idx_blk[b, pl.ds(k * 16, 16)] = cdst[pl.ds(j * 128 + k * 16, 16)]
                return 0

            lax.fori_loop(0, 8, icp, 0)
            pltpu.async_copy(table.at[csrc.at[pl.ds(j * 128, 128)]],
                             rows_v.at[b], sem)

        def drain_block(j, b, sem):
            pltpu.make_async_copy(table.at[csrc.at[pl.ds(j * 128, 128)]],
                                  rows_v.at[b], sem).wait()
            pltpu.sync_copy(rows_v.at[b], acc.at[idx_blk.at[b]], add=True)
            pltpu.sync_copy(ones_v, cacc.at[idx_blk.at[b]], add=True)

        def pair(p, _):
            j0 = p * 2
            j1 = j0 + 1
            stage_block(j0, 0, sem_g0)

            @pl.when(j1 < nb)
            def _():
                stage_block(j1, 1, sem_g1)

            drain_block(j0, 0, sem_g0)

            @pl.when(j1 < nb)
            def _():
                drain_block(j1, 1, sem_g1)

            return 0

        lax.fori_loop(0, (nb + 1) // 2, pair, 0)
        plsc.subcore_barrier()

        r0 = sid * RPT
        pltpu.sync_copy(acc.at[pl.ds(r0, RPT)], pout.at[cid, pl.ds(r0, RPT)])
        pltpu.sync_copy(cacc.at[pl.ds(r0, RPT)], cout.at[cid, pl.ds(r0, RPT)])

    return seg_kernel


_sc_seg0 = _make_sc_segsum(160000 // NW, compact=True)
_sc_seg1 = _make_sc_segsum(32000 // NW, compact=False)


def _tc_body(pref, cref, xref, wlref, wrref, blref, brref, oref, *, act):
    agg = pref[0, :NSEG, :] + pref[1, :NSEG, :]
    cnt = cref[0, :NSEG, 0:1] + cref[1, :NSEG, 0:1]
    mean = agg / jnp.maximum(cnt, 1.0)
    z = (jnp.dot(mean, wlref[...], preferred_element_type=jnp.float32)
         + jnp.dot(xref[...], wrref[...], preferred_element_type=jnp.float32)
         + blref[...] + brref[...])
    if act == "relu":
        oref[...] = jnp.maximum(z, 0.0)
    else:
        m = jnp.max(z, axis=-1, keepdims=True)
        e = jnp.exp(z - m)
        oref[...] = z - m - jnp.log(jnp.sum(e, axis=-1, keepdims=True))


def _tc_layer(p, c, xt, wl, wr, bl, br, act):
    return pl.pallas_call(
        functools.partial(_tc_body, act=act),
        out_shape=jax.ShapeDtypeStruct((NSEG, D), jnp.float32),
    )(p, c, xt, wl, wr, bl.reshape(1, D), br.reshape(1, D))


def kernel(x, edge_index0, edge_index1, W_l0, b_l0, W_r0, b_r0,
           W_l1, b_l1, W_r1, b_r1):
    e0 = edge_index0.astype(jnp.int32).reshape(-1)
    e1 = edge_index1.astype(jnp.int32).reshape(-1)
    zrow = jnp.zeros((RPT, D), jnp.float32)
    ones_h = jnp.ones((128, D), jnp.float32)

    p0, c0 = _sc_seg0(e0, x, zrow, ones_h)
    h = _tc_layer(p0, c0, x[:NSEG], W_l0, W_r0, b_l0, b_r0, "relu")
    p1, c1 = _sc_seg1(e1, h, zrow, ones_h)
    return _tc_layer(p1, c1, h, W_l1, W_r1, b_l1, b_r1, "logsoftmax")
